# baseline (speedup 1.0000x reference)
import jax
import jax.numpy as jnp
from jax.experimental import pallas as pl
from jax.experimental.pallas import tpu as pltpu

EMB = 768
VDIM = 64
VNUM = 1024
NCB = 3
BETA = 0.5
BATCH = 16384
BT = 512
NT = BATCH // BT

_bf = lambda a: a.astype(jnp.bfloat16)


def _dot_bf(a, b):
    return jax.lax.dot_general(_bf(a), _bf(b), (((1,), (0,)), ((), ())),
                               preferred_element_type=jnp.float32)


def _dot_c8(a, b):
    acc = None
    for c in range(0, a.shape[1], 8):
        p = _dot_bf(a[:, c:c + 8], b[c:c + 8])
        acc = p if acc is None else acc + p
    return acc


def _body(x_ref,
          eW0, eb0, eW1, eb1, eW2, eb2, eW3, eb3,
          dW0, db0, dW1, db1, dW2, db2, dW3, db3,
          cb_ref, cbT_ref, cbs_ref,
          quant_ref, idx_ref, qloss_ref, rloss_ref):
    i = pl.program_id(0)

    x = x_ref[...]
    h = jnp.maximum(_dot_bf(x, eW0[...]) + eb0[...], 0.0)
    h = jnp.maximum(_dot_c8(h, eW1[...]) + eb1[...], 0.0)
    h = jnp.maximum(_dot_c8(h, eW2[...]) + eb2[...], 0.0)
    z = _dot_c8(h, eW3[...]) + eb3[...]

    r = z
    quant = jnp.zeros_like(z)
    sse_q = jnp.float32(0.0)
    lane = jax.lax.broadcasted_iota(jnp.int32, (BT, VNUM), 1)
    for c in range(NCB):
        cbT = cbT_ref[c]
        cb2 = jnp.sum(cbT * cbT, axis=0, keepdims=True)
        r2 = jnp.sum(r * r, axis=-1, keepdims=True)
        s = (r2 - 2.0 * _dot_c8(r, cbT)) + cb2
        smin = jnp.min(s, axis=-1, keepdims=True)
        idx = jnp.min(jnp.where(s <= smin, lane, VNUM), axis=-1,
                      keepdims=True)
        onehot = (lane == idx).astype(jnp.bfloat16)
        q = _dot_bf(onehot, cb_ref[c])
        q = (q + _dot_bf(onehot, cbs_ref[0, c])) + _dot_bf(onehot, cbs_ref[1, c])
        diff = r - q
        sse_q += jnp.sum(diff * diff)
        quant += q
        r = diff
        idx_ref[:, c:c + 1] = idx

    quant_ref[...] = quant

    h = jnp.maximum(_dot_bf(quant, dW0[...]) + db0[...], 0.0)
    h = jnp.maximum(_dot_bf(h, dW1[...]) + db1[...], 0.0)
    h = jnp.maximum(_dot_bf(h, dW2[...]) + db2[...], 0.0)
    recon = _dot_bf(h, dW3[...]) + db3[...]
    rd = recon - x
    sse_r = jnp.sum(rd * rd)

    @pl.when(i == 0)
    def _():
        qloss_ref[0, 0] = jnp.float32(0.0)
        rloss_ref[0, 0] = jnp.float32(0.0)

    qacc = qloss_ref[0, 0] + sse_q
    racc = rloss_ref[0, 0] + sse_r
    qloss_ref[0, 0] = qacc
    rloss_ref[0, 0] = racc

    @pl.when(i == NT - 1)
    def _():
        qloss_ref[0, 0] = qacc * ((1.0 + BETA) / (BATCH * VDIM))
        rloss_ref[0, 0] = racc * (1.0 / (BATCH * EMB))


@jax.jit
def kernel(x, enc_W0, enc_b0, enc_W1, enc_b1, enc_W2, enc_b2, enc_W3, enc_b3,
           dec_W0, dec_b0, dec_W1, dec_b1, dec_W2, dec_b2, dec_W3, dec_b3,
           codebooks):
    full = lambda a: pl.BlockSpec(a.shape, lambda i: (0,) * a.ndim)
    biases = [enc_b0, enc_b1, enc_b2, enc_b3, dec_b0, dec_b1, dec_b2, dec_b3]
    ws = [enc_W0, enc_W1, enc_W2, enc_W3, dec_W0, dec_W1, dec_W2, dec_W3]

    in_specs = [pl.BlockSpec((BT, EMB), lambda i: (i, 0))]
    operands = [x]
    for k in range(8):
        in_specs.append(full(ws[k]))
        operands.append(ws[k])
        b2 = biases[k].reshape(1, -1)
        in_specs.append(full(b2))
        operands.append(b2)
    cbT = codebooks.transpose(0, 2, 1)
    cb_hi = codebooks.astype(jnp.bfloat16)
    rem = codebooks - cb_hi.astype(jnp.float32)
    cb_mid = rem.astype(jnp.bfloat16)
    cb_lo = (rem - cb_mid.astype(jnp.float32)).astype(jnp.bfloat16)
    cbs = jnp.stack([cb_mid, cb_lo])
    for a in (codebooks, cbT, cbs):
        in_specs.append(full(a))
        operands.append(a)

    out_shape = (
        jax.ShapeDtypeStruct((BATCH, VDIM), jnp.float32),
        jax.ShapeDtypeStruct((BATCH, NCB), jnp.int32),
        jax.ShapeDtypeStruct((1, 1), jnp.float32),
        jax.ShapeDtypeStruct((1, 1), jnp.float32),
    )
    out_specs = (
        pl.BlockSpec((BT, VDIM), lambda i: (i, 0)),
        pl.BlockSpec((BT, NCB), lambda i: (i, 0)),
        pl.BlockSpec(memory_space=pltpu.SMEM),
        pl.BlockSpec(memory_space=pltpu.SMEM),
    )

    quant, idxs, qloss, rloss = pl.pallas_call(
        _body,
        grid=(NT,),
        in_specs=in_specs,
        out_specs=out_specs,
        out_shape=out_shape,
    )(*operands)
    return quant, qloss[0, 0], rloss[0, 0], idxs
